# Initial kernel scaffold; baseline (speedup 1.0000x reference)
import jax
import jax.numpy as jnp
from jax.experimental import pallas as pl


def kernel(x, edge_index, batch, y_feat, W1, b1, gamma, beta, Wl1, bl1, Wl2, bl2):
    raise NotImplementedError("write your pallas kernel here")


# speedup vs baseline: 88.7469x; 88.7469x over previous
import functools

import jax
import jax.numpy as jnp
from jax import lax
from jax.experimental import pallas as pl
from jax.experimental.pallas import tpu as pltpu
from jax.experimental.pallas import tpu_sc as plsc

N = 10000
E = 320000
G = 64
H = 256
ADD = 7
EPS = 1e-5

NC = 2
NS = 16
NW = NC * NS
WIN = 128
K = -(-E // (NW * WIN))
E_PAD = NW * K * WIN
N_PAD = 10240
SLICE = N_PAD // NS


def _mesh():
    return plsc.VectorSubcoreMesh(
        core_axis_name="c", subcore_axis_name="s", num_cores=NC, num_subcores=NS
    )


def _fill(ref, n, value):
    v = jnp.full((16,), value, jnp.float32)

    @pl.loop(0, n // 16)
    def _(i):
        ref[pl.ds(i * 16, 16)] = v


def _deg_kernel(dst_hbm, out_hbm, acc, dst_v, ones_v, zbuf):
    c = lax.axis_index("c")
    s = lax.axis_index("s")
    wid = s * NC + c

    _fill(zbuf, SLICE, 0.0)
    _fill(ones_v, WIN, 1.0)
    pltpu.sync_copy(zbuf, acc.at[pl.ds(s * SLICE, SLICE)])
    pltpu.sync_copy(dst_hbm.at[wid], dst_v)
    plsc.subcore_barrier()

    @pl.loop(0, K)
    def _(j):
        pltpu.sync_copy(ones_v, acc.at[dst_v.at[j]], add=True)

    plsc.subcore_barrier()
    pltpu.sync_copy(
        acc.at[pl.ds(s * SLICE, SLICE)], out_hbm.at[c, pl.ds(s * SLICE, SLICE)]
    )


def _agg_kernel(src_hbm, dst_hbm, u_hbm, out_hbm, acc, u_sp, src_v, dst_v, vals, zbuf):
    c = lax.axis_index("c")
    s = lax.axis_index("s")
    wid = s * NC + c

    _fill(zbuf, SLICE, 0.0)
    pltpu.sync_copy(zbuf, acc.at[pl.ds(s * SLICE, SLICE)])
    pltpu.sync_copy(
        u_hbm.at[pl.ds(s * SLICE, SLICE)], u_sp.at[pl.ds(s * SLICE, SLICE)]
    )
    pltpu.sync_copy(src_hbm.at[wid], src_v)
    pltpu.sync_copy(dst_hbm.at[wid], dst_v)
    plsc.subcore_barrier()

    @pl.loop(0, K)
    def _(j):
        pltpu.sync_copy(u_sp.at[src_v.at[j]], vals)
        pltpu.sync_copy(vals, acc.at[dst_v.at[j]], add=True)

    plsc.subcore_barrier()
    pltpu.sync_copy(
        acc.at[pl.ds(s * SLICE, SLICE)], out_hbm.at[c, pl.ds(s * SLICE, SLICE)]
    )


def _norm_body(parts_ref, x_ref, dinv_ref, u_ref):
    deg = parts_ref[0] + parts_ref[1] + 1.0
    dinv = lax.rsqrt(deg)
    dinv_ref[...] = dinv
    u_ref[...] = x_ref[...] * dinv


def _dot(a, b):
    return jnp.dot(a, b, precision=lax.Precision.HIGHEST,
                   preferred_element_type=jnp.float32)


def _head_body(aggp_ref, u_ref, dinv_ref, brow_ref, bcol_ref, W1c_ref, b1c_ref,
               gamma_ref, beta_ref, Wl1aT_ref, Wl1bT_ref, bl1c_ref, yfT_ref,
               Wl2T_ref, bl2c_ref, out_ref):
    s_row = dinv_ref[...] * (aggp_ref[0] + aggp_ref[1] + u_ref[...])
    h = jax.nn.gelu(W1c_ref[...] * s_row + b1c_ref[...])
    valid_row = (brow_ref[...] < G).astype(jnp.float32)
    hm = h * valid_row
    total = jnp.sum(hm, axis=1, keepdims=True)
    totalsq = jnp.sum(hm * hm, axis=1, keepdims=True)
    iota_g = lax.broadcasted_iota(jnp.int32, (N_PAD, G), 1)
    onehot_t = (bcol_ref[...] == iota_g).astype(jnp.float32)
    sums_t = _dot(h, onehot_t)
    counts = _dot(valid_row, onehot_t)
    mu = total * (1.0 / N)
    var = totalsq * (1.0 / N) - mu * mu
    pooled_t = sums_t / jnp.maximum(counts, 1.0)
    bn_t = (pooled_t - mu) * lax.rsqrt(var + EPS) * gamma_ref[...] + beta_ref[...]
    z1 = _dot(Wl1aT_ref[...], bn_t) + _dot(Wl1bT_ref[...], yfT_ref[...]) + bl1c_ref[...]
    g1 = jax.nn.gelu(z1)
    out_ref[...] = jax.nn.sigmoid(_dot(Wl2T_ref[...], g1) + bl2c_ref[...])


def kernel(x, edge_index, batch, y_feat, W1, b1, gamma, beta, Wl1, bl1, Wl2, bl2):
    f32 = jnp.float32
    src = edge_index[0]
    dst = edge_index[1]
    pad = E_PAD - E
    src_w = jnp.concatenate([src, jnp.zeros((pad,), jnp.int32)]).reshape(NW, K, WIN)
    dst_w = jnp.concatenate([dst, jnp.full((pad,), N, jnp.int32)]).reshape(NW, K, WIN)

    deg_call = pl.kernel(
        _deg_kernel,
        out_type=jax.ShapeDtypeStruct((NC, N_PAD), f32),
        mesh=_mesh(),
        scratch_types=[
            pltpu.VMEM_SHARED((N_PAD,), f32),
            pltpu.VMEM((K, WIN), jnp.int32),
            pltpu.VMEM((WIN,), f32),
            pltpu.VMEM((SLICE,), f32),
        ],
    )
    deg_parts = deg_call(dst_w)

    xp = jnp.pad(x[:, 0], (0, N_PAD - N)).reshape(80, 128)
    dinv2d, u2d = pl.pallas_call(
        _norm_body,
        out_shape=[
            jax.ShapeDtypeStruct((80, 128), f32),
            jax.ShapeDtypeStruct((80, 128), f32),
        ],
    )(deg_parts.reshape(NC, 80, 128), xp)
    u_flat = u2d.reshape(N_PAD)

    agg_call = pl.kernel(
        _agg_kernel,
        out_type=jax.ShapeDtypeStruct((NC, N_PAD), f32),
        mesh=_mesh(),
        scratch_types=[
            pltpu.VMEM_SHARED((N_PAD,), f32),
            pltpu.VMEM_SHARED((N_PAD,), f32),
            pltpu.VMEM((K, WIN), jnp.int32),
            pltpu.VMEM((K, WIN), jnp.int32),
            pltpu.VMEM((WIN,), f32),
            pltpu.VMEM((SLICE,), f32),
        ],
    )
    agg_parts = agg_call(src_w, dst_w, u_flat)

    batch_p = jnp.pad(batch, (0, N_PAD - N), constant_values=G)
    out_t = pl.pallas_call(
        _head_body,
        out_shape=jax.ShapeDtypeStruct((2, G), f32),
    )(
        agg_parts.reshape(NC, 1, N_PAD),
        u_flat.reshape(1, N_PAD),
        dinv2d.reshape(1, N_PAD),
        batch_p.reshape(1, N_PAD),
        batch_p.reshape(N_PAD, 1),
        W1.reshape(H, 1),
        b1.reshape(H, 1),
        gamma.reshape(H, 1),
        beta.reshape(H, 1),
        Wl1[:H].T,
        Wl1[H:].T,
        bl1.reshape(-1, 1),
        y_feat.T,
        Wl2.T,
        bl2.reshape(-1, 1),
    )
    return out_t.T
